# baseline (speedup 1.0000x reference)
import functools

import jax
import jax.numpy as jnp
from jax import lax
from jax.experimental import pallas as pl
from jax.experimental.pallas import tpu as pltpu
from jax.experimental.pallas import tpu_sc as plsc

_TC_ROWS = 3072
_TC_CHUNKS = 4
_SC_CHUNK_ROWS = 16


def _staged_copy(w_ref, o_ref, scratch, in_sems, out_sems):
    rows = o_ref.shape[0]
    chunk = rows // _TC_CHUNKS
    in_copies = [
        pltpu.make_async_copy(
            w_ref.at[pl.ds(i * chunk, chunk), :],
            scratch.at[pl.ds(i * chunk, chunk), :],
            in_sems.at[i],
        )
        for i in range(_TC_CHUNKS)
    ]
    out_copies = [
        pltpu.make_async_copy(
            scratch.at[pl.ds(i * chunk, chunk), :],
            o_ref.at[pl.ds(i * chunk, chunk), :],
            out_sems.at[i],
        )
        for i in range(_TC_CHUNKS)
    ]
    for c in in_copies:
        c.start()
    for i in range(_TC_CHUNKS):
        in_copies[i].wait()
        out_copies[i].start()
    for c in out_copies:
        c.wait()


def _make_sc_copy(row_base, rows, hidden, dtype):
    info = plsc.get_sparse_core_info()
    nc, ns = info.num_cores, info.num_subcores
    nw = nc * ns
    rows_per_w = rows // nw
    n_chunks = max(1, rows_per_w // _SC_CHUNK_ROWS)
    chunk = rows_per_w // n_chunks
    mesh = plsc.VectorSubcoreMesh(core_axis_name="c", subcore_axis_name="s")

    @functools.partial(
        pl.kernel,
        mesh=mesh,
        out_type=jax.ShapeDtypeStruct((rows, hidden), dtype),
        scratch_types=[
            pltpu.VMEM((None, hidden), dtype) if False else pltpu.VMEM((16, hidden), dtype),
            pltpu.VMEM((16, hidden), dtype),
            pltpu.SemaphoreType.DMA,
            pltpu.SemaphoreType.DMA,
            pltpu.SemaphoreType.DMA,
            pltpu.SemaphoreType.DMA,
        ],
    )
    def sc_copy(w_hbm, o_hbm, buf0, buf1, rs0, rs1, ws0, ws1):
        wid = lax.axis_index("s") * nc + lax.axis_index("c")
        base = wid * rows_per_w
        bufs = (buf0, buf1)
        rsems = (rs0, rs1)
        wsems = (ws0, ws1)
        reads = [None] * n_chunks
        writes = [None] * n_chunks
        for c in range(min(2, n_chunks)):
            reads[c] = pltpu.async_copy(
                w_hbm.at[pl.ds(row_base + base + c * chunk, chunk), :],
                bufs[c % 2].at[pl.ds(0, chunk), :],
                rsems[c % 2],
            )
        for c in range(n_chunks):
            b = c % 2
            reads[c].wait()
            writes[c] = pltpu.async_copy(
                bufs[b].at[pl.ds(0, chunk), :],
                o_hbm.at[pl.ds(base + c * chunk, chunk), :],
                wsems[b],
            )
            nxt = c + 2
            if nxt < n_chunks:
                writes[c].wait()
                reads[nxt] = pltpu.async_copy(
                    w_hbm.at[pl.ds(row_base + base + nxt * chunk, chunk), :],
                    bufs[b].at[pl.ds(0, chunk), :],
                    rsems[b],
                )
        for c in range(max(0, n_chunks - 2), n_chunks):
            writes[c].wait()

    return sc_copy


def kernel(x, weight):
    seq_len = x.shape[1]
    hidden = weight.shape[1]
    tc_out = pl.pallas_call(
        _staged_copy,
        in_specs=[pl.BlockSpec(memory_space=pl.ANY)],
        out_specs=pl.BlockSpec(memory_space=pl.ANY),
        out_shape=jax.ShapeDtypeStruct((_TC_ROWS, hidden), weight.dtype),
        scratch_shapes=[
            pltpu.VMEM((_TC_ROWS, hidden), weight.dtype),
            pltpu.SemaphoreType.DMA((_TC_CHUNKS,)),
            pltpu.SemaphoreType.DMA((_TC_CHUNKS,)),
        ],
    )(weight)
    sc_out = _make_sc_copy(_TC_ROWS, seq_len - _TC_ROWS, hidden, weight.dtype)(weight)
    return (tc_out, sc_out)


# speedup vs baseline: 1.8237x; 1.8237x over previous
import jax
import jax.numpy as jnp
from jax.experimental import pallas as pl
from jax.experimental.pallas import tpu as pltpu

_NUM_CHUNKS = 2


def _staged_copy(w_ref, o_ref, scratch, in_sems, out_sems):
    rows = o_ref.shape[0]
    chunk = rows // _NUM_CHUNKS
    in_copies = [
        pltpu.make_async_copy(
            w_ref.at[pl.ds(i * chunk, chunk), :],
            scratch.at[pl.ds(i * chunk, chunk), :],
            in_sems.at[i],
        )
        for i in range(_NUM_CHUNKS)
    ]
    out_copies = [
        pltpu.make_async_copy(
            scratch.at[pl.ds(i * chunk, chunk), :],
            o_ref.at[pl.ds(i * chunk, chunk), :],
            out_sems.at[i],
        )
        for i in range(_NUM_CHUNKS)
    ]
    for c in in_copies:
        c.start()
    for i in range(_NUM_CHUNKS):
        in_copies[i].wait()
        out_copies[i].start()
    for c in out_copies:
        c.wait()


def kernel(x, weight):
    seq_len = x.shape[1]
    hidden = weight.shape[1]
    out = pl.pallas_call(
        _staged_copy,
        in_specs=[pl.BlockSpec(memory_space=pl.ANY)],
        out_specs=pl.BlockSpec(memory_space=pl.ANY),
        out_shape=jax.ShapeDtypeStruct((seq_len, hidden), weight.dtype),
        scratch_shapes=[
            pltpu.VMEM((seq_len, hidden), weight.dtype),
            pltpu.SemaphoreType.DMA((_NUM_CHUNKS,)),
            pltpu.SemaphoreType.DMA((_NUM_CHUNKS,)),
        ],
    )(weight)
    return out[None, :, :]
